# baseline (speedup 1.0000x reference)
import jax
import jax.numpy as jnp
from jax import lax
from jax.experimental import pallas as pl
from jax.experimental.pallas import tpu as pltpu
from jax.experimental.pallas import tpu_sc as plsc

_VOCAB = 1000000
_HIDDEN = 64
_PAD = 128
_BATCH = 16384
_HIST = 200

_NC = 2
_NS = 16
_NW = _NC * _NS
_BC = 256
_NBC = _BATCH // _BC
_NTASK = _HIST * _NBC
_TPW = _NTASK // _NW
_KT = _HIDDEN // 8


def _body(xT_hbm, w_hbm, out_hbm, idx0, idx1, g0, g1, t_v, sg0, sg1, sw):
    wid = lax.axis_index("s") * _NC + lax.axis_index("c")
    t0 = wid * _TPW

    def start_gather(idx_v, g_v, sg, t):
        h = t // _NBC
        b0 = (t % _NBC) * _BC
        pltpu.sync_copy(xT_hbm.at[h, pl.ds(b0, _BC)], idx_v)
        pltpu.async_copy(w_hbm.at[idx_v], g_v, sg)

    def wait_gather(idx_v, g_v, sg):
        pltpu.make_async_copy(w_hbm.at[idx_v], g_v, sg).wait()

    def start_writes(t):
        h = t // _NBC
        b0 = (t % _NBC) * _BC
        pltpu.async_copy(t_v, out_hbm.at[h, :, pl.ds(b0, _BC)], sw)

    def wait_writes(t):
        h = t // _NBC
        b0 = (t % _NBC) * _BC
        pltpu.make_async_copy(t_v, out_hbm.at[h, :, pl.ds(b0, _BC)],
                              sw).wait()

    def xpose(g_v):
        kvecs = [lax.iota(jnp.int32, 16) + 16 * c for c in range(_HIDDEN // 16)]

        @plsc.parallel_loop(0, _BC, step=1, unroll=8)
        def row_fn(j):
            row = g_v.at[j]
            jvec = jnp.full((16,), 0, jnp.int32) + j
            for c in range(_HIDDEN // 16):
                plsc.store_scatter(t_v, [kvecs[c], jvec],
                                   row[pl.ds(16 * c, 16)])

    start_gather(idx0, g0, sg0, t0)
    start_gather(idx1, g1, sg1, t0 + 1)

    def step(i, carry):
        t = t0 + i * 2

        wait_gather(idx0, g0, sg0)

        @pl.when(i > 0)
        def _():
            wait_writes(t - 1)
        xpose(g0)
        start_writes(t)

        @pl.when(i * 2 + 2 < _TPW)
        def _():
            start_gather(idx0, g0, sg0, t + 2)

        wait_gather(idx1, g1, sg1)
        wait_writes(t)
        xpose(g1)
        start_writes(t + 1)

        @pl.when(i * 2 + 3 < _TPW)
        def _():
            start_gather(idx1, g1, sg1, t + 3)

        return carry

    lax.fori_loop(0, _TPW // 2, step, 0)
    wait_writes(t0 + _TPW - 1)


def kernel(x, weight):
    xT = jnp.transpose(x).astype(jnp.int32)
    wp = jnp.pad(weight, ((0, 0), (0, _PAD - _HIDDEN)))
    mesh = plsc.VectorSubcoreMesh(
        core_axis_name="c", subcore_axis_name="s",
        num_cores=_NC, num_subcores=_NS)
    outP = pl.kernel(
        _body,
        out_type=jax.ShapeDtypeStruct((_HIST, _HIDDEN, _BATCH), jnp.float32),
        mesh=mesh,
        compiler_params=pltpu.CompilerParams(use_tc_tiling_on_sc=True,
                                             needs_layout_passes=False),
        scratch_types=[
            pltpu.VMEM((_BC,), jnp.int32),
            pltpu.VMEM((_BC,), jnp.int32),
            pltpu.VMEM((_BC, _PAD), jnp.float32),
            pltpu.VMEM((_BC, _PAD), jnp.float32),
            pltpu.VMEM((_HIDDEN, _BC), jnp.float32),
            pltpu.SemaphoreType.DMA,
            pltpu.SemaphoreType.DMA,
            pltpu.SemaphoreType.DMA,
        ],
    )(xT, wp)
    return jnp.transpose(outP, (2, 0, 1))


# speedup vs baseline: 1.0030x; 1.0030x over previous
import jax
import jax.numpy as jnp
from jax import lax
from jax.experimental import pallas as pl
from jax.experimental.pallas import tpu as pltpu
from jax.experimental.pallas import tpu_sc as plsc

_VOCAB = 1000000
_HIDDEN = 64
_PAD = 128
_BATCH = 16384
_HIST = 200

_NC = 2
_NS = 16
_NW = _NC * _NS
_BC = 256
_NBC = _BATCH // _BC
_NTASK = _HIST * _NBC
_TPW = _NTASK // _NW
_KT = _HIDDEN // 8


def _body(xT_hbm, w_hbm, out_hbm, idx0, idx1, g0, g1, t_v, sg0, sg1, sw):
    wid = lax.axis_index("s") * _NC + lax.axis_index("c")
    t0 = wid * _TPW

    def start_gather(idx_v, g_v, sg, t):
        h = t // _NBC
        b0 = (t % _NBC) * _BC
        pltpu.sync_copy(xT_hbm.at[h, pl.ds(b0, _BC)], idx_v)
        pltpu.async_copy(w_hbm.at[idx_v], g_v, sg)

    def wait_gather(idx_v, g_v, sg):
        pltpu.make_async_copy(w_hbm.at[idx_v], g_v, sg).wait()

    def start_writes(t):
        h = t // _NBC
        b0 = (t % _NBC) * _BC
        pltpu.async_copy(t_v.at[:, pl.ds(0, _BC)],
                         out_hbm.at[h, :, pl.ds(b0, _BC)], sw)

    def wait_writes(t):
        h = t // _NBC
        b0 = (t % _NBC) * _BC
        pltpu.make_async_copy(t_v.at[:, pl.ds(0, _BC)],
                              out_hbm.at[h, :, pl.ds(b0, _BC)], sw).wait()

    def xpose(g_v):
        kvecs = [lax.iota(jnp.int32, 16) + 16 * c for c in range(_HIDDEN // 16)]

        @plsc.parallel_loop(0, _BC, step=1, unroll=8)
        def row_fn(j):
            row = g_v.at[j]
            jvec = jnp.full((16,), 0, jnp.int32) + j
            for c in range(_HIDDEN // 16):
                plsc.store_scatter(t_v, [kvecs[c], jvec],
                                   row[pl.ds(16 * c, 16)])

    start_gather(idx0, g0, sg0, t0)
    start_gather(idx1, g1, sg1, t0 + 1)

    def step(i, carry):
        t = t0 + i * 2

        wait_gather(idx0, g0, sg0)

        @pl.when(i > 0)
        def _():
            wait_writes(t - 1)
        xpose(g0)
        start_writes(t)

        @pl.when(i * 2 + 2 < _TPW)
        def _():
            start_gather(idx0, g0, sg0, t + 2)

        wait_gather(idx1, g1, sg1)
        wait_writes(t)
        xpose(g1)
        start_writes(t + 1)

        @pl.when(i * 2 + 3 < _TPW)
        def _():
            start_gather(idx1, g1, sg1, t + 3)

        return carry

    lax.fori_loop(0, _TPW // 2, step, 0)
    wait_writes(t0 + _TPW - 1)


def kernel(x, weight):
    xT = jnp.transpose(x).astype(jnp.int32)
    wp = jnp.pad(weight, ((0, 0), (0, _PAD - _HIDDEN)))
    mesh = plsc.VectorSubcoreMesh(
        core_axis_name="c", subcore_axis_name="s",
        num_cores=_NC, num_subcores=_NS)
    outP = pl.kernel(
        _body,
        out_type=jax.ShapeDtypeStruct((_HIST, _HIDDEN, _BATCH), jnp.float32),
        mesh=mesh,
        compiler_params=pltpu.CompilerParams(use_tc_tiling_on_sc=True,
                                             needs_layout_passes=False),
        scratch_types=[
            pltpu.VMEM((_BC,), jnp.int32),
            pltpu.VMEM((_BC,), jnp.int32),
            pltpu.VMEM((_BC, _PAD), jnp.float32),
            pltpu.VMEM((_BC, _PAD), jnp.float32),
            pltpu.VMEM((_HIDDEN, _BC + 1), jnp.float32),
            pltpu.SemaphoreType.DMA,
            pltpu.SemaphoreType.DMA,
            pltpu.SemaphoreType.DMA,
        ],
    )(xT, wp)
    return jnp.transpose(outP, (2, 0, 1))


# speedup vs baseline: 1.1424x; 1.1390x over previous
import jax
import jax.numpy as jnp
from jax import lax
from jax.experimental import pallas as pl
from jax.experimental.pallas import tpu as pltpu
from jax.experimental.pallas import tpu_sc as plsc

_VOCAB = 1000000
_HIDDEN = 64
_BATCH = 16384
_HIST = 200
_B = _BATCH * _HIST

_NC = 2
_NS = 16
_NW = _NC * _NS
_BPW = _B // _NW
_C = 800
_NCHUNK = _BPW // _C


def _body(x_hbm, w_hbm, out_hbm,
          idx0, idx1, rows0, rows1, sg0, sg1, sw0, sw1):
    wid = lax.axis_index("s") * _NC + lax.axis_index("c")
    base = wid * _BPW

    def start_chunk(idx_v, rows_v, sg, c):
        pltpu.sync_copy(x_hbm.at[pl.ds(base + c * _C, _C)], idx_v)
        pltpu.async_copy(w_hbm.at[idx_v], rows_v, sg)

    def wait_gather(idx_v, rows_v, sg):
        pltpu.make_async_copy(w_hbm.at[idx_v], rows_v, sg).wait()

    def start_write(rows_v, sw, c):
        pltpu.async_copy(rows_v, out_hbm.at[pl.ds(base + c * _C, _C)], sw)

    def wait_write(rows_v, sw, c):
        pltpu.make_async_copy(rows_v, out_hbm.at[pl.ds(base + c * _C, _C)],
                              sw).wait()

    start_chunk(idx0, rows0, sg0, 0)
    start_chunk(idx1, rows1, sg1, 1)

    def step(j, carry):
        c0 = j * 2

        wait_gather(idx0, rows0, sg0)
        start_write(rows0, sw0, c0)

        @pl.when(c0 + 2 < _NCHUNK)
        def _():
            wait_write(rows0, sw0, c0)
            start_chunk(idx0, rows0, sg0, c0 + 2)

        wait_gather(idx1, rows1, sg1)
        start_write(rows1, sw1, c0 + 1)

        @pl.when(c0 + 3 < _NCHUNK)
        def _():
            wait_write(rows1, sw1, c0 + 1)
            start_chunk(idx1, rows1, sg1, c0 + 3)

        return carry

    lax.fori_loop(0, _NCHUNK // 2, step, 0)

    wait_write(rows0, sw0, _NCHUNK - 2)
    wait_write(rows1, sw1, _NCHUNK - 1)


def kernel(x, weight):
    xf = x.reshape(-1).astype(jnp.int32)
    mesh = plsc.VectorSubcoreMesh(
        core_axis_name="c", subcore_axis_name="s",
        num_cores=_NC, num_subcores=_NS)
    out = pl.kernel(
        _body,
        out_type=jax.ShapeDtypeStruct((_B, _HIDDEN), jnp.float32),
        mesh=mesh,
        compiler_params=pltpu.CompilerParams(use_tc_tiling_on_sc=False),
        scratch_types=[
            pltpu.VMEM((_C,), jnp.int32),
            pltpu.VMEM((_C,), jnp.int32),
            pltpu.VMEM((_C, _HIDDEN), jnp.float32),
            pltpu.VMEM((_C, _HIDDEN), jnp.float32),
            pltpu.SemaphoreType.DMA,
            pltpu.SemaphoreType.DMA,
            pltpu.SemaphoreType.DMA,
            pltpu.SemaphoreType.DMA,
        ],
    )(xf, weight)
    return out.reshape(_BATCH, _HIST, _HIDDEN)
